# baseline (speedup 1.0000x reference)
import functools

import jax
import jax.numpy as jnp
from jax import lax
from jax.experimental import pallas as pl
from jax.experimental.pallas import tpu as pltpu
from jax.experimental.pallas import tpu_sc as plsc

_N = 100000
_D = 64
_B = 16384
_MARGIN = 1.0
_BETA1 = 0.1

_NC = 2
_NS = 16
_NW = _NC * _NS
_BPW = _B // _NW
_CHUNK = 128
_NCHUNK = _BPW // _CHUNK
_L = 16
_NGROUP = _BPW // _L


def _hinge_body(emb_hbm, jkn_hbm, out_hbm, idx_v, rows_j, rows_k, rows_n,
                hsum_v, sem):
    wid = lax.axis_index("s") * _NC + lax.axis_index("c")
    pltpu.sync_copy(jkn_hbm.at[wid], idx_v)
    descs = []
    for t, rows in enumerate((rows_j, rows_k, rows_n)):
        for c in range(_NCHUNK):
            descs.append(
                pltpu.async_copy(emb_hbm.at[idx_v.at[t, c]],
                                 rows.at[pl.ds(c * _CHUNK, _CHUNK)], sem))
    for d in descs:
        d.wait()

    lanes = lax.broadcasted_iota(jnp.int32, (_L,), 0)

    def group(g, hsum):
        samp = g * _L + lanes
        delta = jnp.zeros((_L,), jnp.float32)
        for d in range(_D):
            col = jnp.full((_L,), d, jnp.int32)
            gj = plsc.load_gather(rows_j, [samp, col])
            gk = plsc.load_gather(rows_k, [samp, col])
            gn = plsc.load_gather(rows_n, [samp, col])
            a = gj - gk
            b = gn - gk
            delta = delta + (a * a - b * b)
        return hsum + jnp.maximum(delta + _MARGIN, 0.0)

    hsum = lax.fori_loop(0, _NGROUP, group, jnp.zeros((_L,), jnp.float32))
    hsum_v[...] = hsum
    pltpu.sync_copy(hsum_v, out_hbm.at[wid])


@jax.jit
def _hinge(embeddings, jkn):
    mesh = plsc.VectorSubcoreMesh(core_axis_name="c", subcore_axis_name="s")
    return pl.kernel(
        _hinge_body,
        out_type=jax.ShapeDtypeStruct((_NW, _L), jnp.float32),
        mesh=mesh,
        scratch_types=[
            pltpu.VMEM((3, _NCHUNK, _CHUNK), jnp.int32),
            pltpu.VMEM((_BPW, _D), jnp.float32),
            pltpu.VMEM((_BPW, _D), jnp.float32),
            pltpu.VMEM((_BPW, _D), jnp.float32),
            pltpu.VMEM((_L,), jnp.float32),
            pltpu.SemaphoreType.DMA,
        ],
        compiler_params=pltpu.CompilerParams(needs_layout_passes=False,
                                             use_tc_tiling_on_sc=False),
    )(embeddings, jkn)


_SROWS = 5000


def _smooth_body(e_ref, l_ref, out_ref):
    i = pl.program_id(0)
    d = e_ref[...] - l_ref[...]
    s = jnp.sum(d * d)

    @pl.when(i == 0)
    def _():
        out_ref[0, 0] = s

    @pl.when(i > 0)
    def _():
        out_ref[0, 0] += s


@jax.jit
def _smooth(e2, l2):
    grid = e2.shape[0] // _SROWS
    return pl.pallas_call(
        _smooth_body,
        grid=(grid,),
        in_specs=[
            pl.BlockSpec((_SROWS, 128), lambda i: (i, 0)),
            pl.BlockSpec((_SROWS, 128), lambda i: (i, 0)),
        ],
        out_specs=pl.BlockSpec(memory_space=pltpu.SMEM),
        out_shape=jax.ShapeDtypeStruct((1, 1), jnp.float32),
    )(e2, l2)


def kernel(embeddings, last_embeddings, triplets):
    trip = triplets.astype(jnp.int32)
    jkn = trip.T.reshape(3, _NW, _NCHUNK, _CHUNK).transpose(1, 0, 2, 3)
    parts = _hinge(embeddings, jkn)
    sm = _smooth(embeddings.reshape(-1, 128), last_embeddings.reshape(-1, 128))
    return jnp.sum(parts) + _BETA1 * (_B * sm[0, 0])


# speedup vs baseline: 1.6043x; 1.6043x over previous
import functools

import jax
import jax.numpy as jnp
from jax import lax
from jax.experimental import pallas as pl
from jax.experimental.pallas import tpu as pltpu
from jax.experimental.pallas import tpu_sc as plsc

_N = 100000
_D = 64
_B = 16384
_MARGIN = 1.0
_BETA1 = 0.1

_NC = 2
_NS = 16
_FPW = _D // (_NC * _NS)
_L = 16
_SCHUNK = 4096
_NSCHUNK = _B // _SCHUNK
_GPC = _SCHUNK // _L
_ROWS = _B // _L
_RPC = _GPC
_UNROLL = 4


def _delta_body(et_hbm, jkn_hbm, out_hbm,
                row_v, idx_v, contrib_v, rowidx_v, zrow_v, delta_sh):
    cid = lax.axis_index("c")
    sid = lax.axis_index("s")
    lanes = lax.broadcasted_iota(jnp.int32, (_L,), 0)
    zero = jnp.zeros((_L,), jnp.float32)

    for r in range(64):
        zrow_v[r] = zero
    pltpu.sync_copy(zrow_v, delta_sh.at[pl.ds(sid * 64, 64)])

    for c in range(_NSCHUNK):
        for h in range(2):
            base = c * _RPC + h * 128
            for i in range(8):
                rowidx_v[c, h, pl.ds(i * _L, _L)] = base + i * _L + lanes

    plsc.subcore_barrier()

    def do_feature(f):
        pltpu.sync_copy(et_hbm.at[f], row_v)
        for c in range(_NSCHUNK):
            for t in range(3):
                pltpu.sync_copy(jkn_hbm.at[t, c], idx_v.at[t])

            def group(g4, carry):
                for u in range(_UNROLL):
                    g = g4 * _UNROLL + u
                    jv = idx_v[0, pl.ds(g * _L, _L)]
                    kv = idx_v[1, pl.ds(g * _L, _L)]
                    nv = idx_v[2, pl.ds(g * _L, _L)]
                    gj = plsc.load_gather(row_v, [jv])
                    gk = plsc.load_gather(row_v, [kv])
                    gn = plsc.load_gather(row_v, [nv])
                    a = gj - gk
                    b = gn - gk
                    contrib_v[g] = a * a - b * b
                return carry

            lax.fori_loop(0, _GPC // _UNROLL, group, jnp.int32(0))
            for h in range(2):
                pltpu.sync_copy(contrib_v.at[pl.ds(h * 128, 128)],
                                delta_sh.at[rowidx_v.at[c, h]], add=True)

    f0 = cid * (_NS * _FPW) + sid * _FPW
    for k in range(_FPW):
        do_feature(f0 + k)

    plsc.subcore_barrier()

    @pl.when(sid == 0)
    def _():
        pltpu.sync_copy(delta_sh, out_hbm.at[cid])


@jax.jit
def _delta(et, jkn):
    mesh = plsc.VectorSubcoreMesh(core_axis_name="c", subcore_axis_name="s")
    return pl.kernel(
        _delta_body,
        out_type=jax.ShapeDtypeStruct((_NC, _ROWS, _L), jnp.float32),
        mesh=mesh,
        scratch_types=[
            pltpu.VMEM((_N,), jnp.float32),
            pltpu.VMEM((3, _SCHUNK), jnp.int32),
            pltpu.VMEM((_GPC, _L), jnp.float32),
            pltpu.VMEM((_NSCHUNK, 2, 128), jnp.int32),
            pltpu.VMEM((64, _L), jnp.float32),
            pltpu.VMEM_SHARED((_ROWS, _L), jnp.float32),
        ],
        compiler_params=pltpu.CompilerParams(use_tc_tiling_on_sc=False,
                                             needs_layout_passes=False),
    )(et, jkn)


_SROWS = 8


def _smooth_body(e_ref, l_ref, out_ref):
    i = pl.program_id(0)
    d = e_ref[...] - l_ref[...]
    s = jnp.sum(d * d)

    @pl.when(i == 0)
    def _():
        out_ref[0, 0] = s

    @pl.when(i > 0)
    def _():
        out_ref[0, 0] += s


@jax.jit
def _smooth(e2, l2):
    grid = e2.shape[0] // _SROWS
    return pl.pallas_call(
        _smooth_body,
        grid=(grid,),
        in_specs=[
            pl.BlockSpec((_SROWS, _N), lambda i: (i, 0)),
            pl.BlockSpec((_SROWS, _N), lambda i: (i, 0)),
        ],
        out_specs=pl.BlockSpec(memory_space=pltpu.SMEM),
        out_shape=jax.ShapeDtypeStruct((1, 1), jnp.float32),
    )(e2, l2)


def _fin_body(dp_ref, sm_ref, out_ref):
    d = dp_ref[0:1, :] + dp_ref[1:2, :]
    h = jnp.maximum(d + _MARGIN, 0.0)
    out_ref[0, 0] = jnp.sum(h) + _BETA1 * (float(_B) * sm_ref[0, 0])


@jax.jit
def _fin(dp, sm):
    return pl.pallas_call(
        _fin_body,
        in_specs=[
            pl.BlockSpec((_NC, _B), lambda: (0, 0)),
            pl.BlockSpec(memory_space=pltpu.SMEM),
        ],
        out_specs=pl.BlockSpec(memory_space=pltpu.SMEM),
        out_shape=jax.ShapeDtypeStruct((1, 1), jnp.float32),
    )(dp, sm)


def kernel(embeddings, last_embeddings, triplets):
    trip = triplets.astype(jnp.int32)
    jkn = trip.T.reshape(3, _NSCHUNK, _SCHUNK)
    dp = _delta(embeddings.T, jkn)
    sm = _smooth(embeddings.T, last_embeddings.T)
    return _fin(dp.reshape(_NC, _B), sm)[0, 0]


# speedup vs baseline: 1.6799x; 1.0471x over previous
import jax
import jax.numpy as jnp
from jax import lax
from jax.experimental import pallas as pl
from jax.experimental.pallas import tpu as pltpu
from jax.experimental.pallas import tpu_sc as plsc

_N = 100000
_D = 64
_B = 16384
_MARGIN = 1.0
_BETA1 = 0.1

_NC = 2
_NS = 16
_NW = _NC * _NS
_L = 16
_BPW = _B // _NW
_CHK = _D // _L
_UNROLL = 4


def _delta_body(et_hbm, idx_hbm, out_hbm,
                idx_v, uj_v, uk_v, un_v, part_v, sem):
    cid = lax.axis_index("c")
    sid = lax.axis_index("s")
    wid = sid * _NC + cid
    base = wid * _BPW

    for t in range(3):
        pltpu.sync_copy(idx_hbm.at[t, pl.ds(base, _BPW)], idx_v.at[t])

    cj = pltpu.async_copy(et_hbm.at[idx_v.at[0]], uj_v, sem)
    ck = pltpu.async_copy(et_hbm.at[idx_v.at[1]], uk_v, sem)
    cn = pltpu.async_copy(et_hbm.at[idx_v.at[2]], un_v, sem)
    cj.wait()
    ck.wait()
    cn.wait()

    def sample(s4, carry):
        for u in range(_UNROLL):
            s = s4 * _UNROLL + u
            acc = jnp.zeros((_L,), jnp.float32)
            for c in range(_CHK):
                uj = uj_v[s, pl.ds(c * _L, _L)]
                uk = uk_v[s, pl.ds(c * _L, _L)]
                un = un_v[s, pl.ds(c * _L, _L)]
                d = uj - un
                m = uj + un - uk - uk
                acc = acc + d * m
            part_v[s] = acc
        return carry

    lax.fori_loop(0, _BPW // _UNROLL, sample, jnp.int32(0))

    pltpu.sync_copy(part_v, out_hbm.at[pl.ds(base, _BPW)])


@jax.jit
def _delta(et, idx):
    mesh = plsc.VectorSubcoreMesh(core_axis_name="c", subcore_axis_name="s")
    return pl.kernel(
        _delta_body,
        out_type=jax.ShapeDtypeStruct((_B, _L), jnp.float32),
        mesh=mesh,
        scratch_types=[
            pltpu.VMEM((3, _BPW), jnp.int32),
            pltpu.VMEM((_BPW, _D), jnp.float32),
            pltpu.VMEM((_BPW, _D), jnp.float32),
            pltpu.VMEM((_BPW, _D), jnp.float32),
            pltpu.VMEM((_BPW, _L), jnp.float32),
            pltpu.SemaphoreType.DMA,
        ],
        compiler_params=pltpu.CompilerParams(use_tc_tiling_on_sc=False,
                                             needs_layout_passes=False),
    )(et, idx)


_SROWS = 8


def _smooth_body(e_ref, l_ref, out_ref):
    i = pl.program_id(0)
    d = e_ref[...] - l_ref[...]
    s = jnp.sum(d * d)

    @pl.when(i == 0)
    def _():
        out_ref[0, 0] = s

    @pl.when(i > 0)
    def _():
        out_ref[0, 0] += s


@jax.jit
def _smooth(e2, l2):
    grid = e2.shape[0] // _SROWS
    return pl.pallas_call(
        _smooth_body,
        grid=(grid,),
        in_specs=[
            pl.BlockSpec((_SROWS, _N), lambda i: (i, 0)),
            pl.BlockSpec((_SROWS, _N), lambda i: (i, 0)),
        ],
        out_specs=pl.BlockSpec(memory_space=pltpu.SMEM),
        out_shape=jax.ShapeDtypeStruct((1, 1), jnp.float32),
    )(e2, l2)


def _fin_body(dp_ref, sm_ref, out_ref):
    d = jnp.sum(dp_ref[...], axis=1)
    h = jnp.maximum(d + _MARGIN, 0.0)
    out_ref[0, 0] = jnp.sum(h) + _BETA1 * (float(_B) * sm_ref[0, 0])


@jax.jit
def _fin(dp, sm):
    return pl.pallas_call(
        _fin_body,
        in_specs=[
            pl.BlockSpec((_B, _L), lambda: (0, 0)),
            pl.BlockSpec(memory_space=pltpu.SMEM),
        ],
        out_specs=pl.BlockSpec(memory_space=pltpu.SMEM),
        out_shape=jax.ShapeDtypeStruct((1, 1), jnp.float32),
    )(dp, sm)


def kernel(embeddings, last_embeddings, triplets):
    idx = triplets.astype(jnp.int32).T
    dp = _delta(embeddings, idx)
    sm = _smooth(embeddings.T, last_embeddings.T)
    return _fin(dp, sm)[0, 0]


# speedup vs baseline: 1.6803x; 1.0002x over previous
import jax
import jax.numpy as jnp
from jax import lax
from jax.experimental import pallas as pl
from jax.experimental.pallas import tpu as pltpu
from jax.experimental.pallas import tpu_sc as plsc

_N = 100000
_D = 64
_B = 16384
_MARGIN = 1.0
_BETA1 = 0.1

_NC = 2
_NS = 16
_NW = _NC * _NS
_L = 16
_BPW = _B // _NW
_CHK = _D // _L
_UNROLL = 4


def _delta_body(et_hbm, idx_hbm, out_hbm,
                idx_v, uj_v, uk_v, un_v, part_v, sem):
    cid = lax.axis_index("c")
    sid = lax.axis_index("s")
    wid = sid * _NC + cid
    base = wid * _BPW

    for t in range(3):
        pltpu.sync_copy(idx_hbm.at[t, pl.ds(base, _BPW)], idx_v.at[t])

    cj = pltpu.async_copy(et_hbm.at[idx_v.at[0]], uj_v, sem)
    ck = pltpu.async_copy(et_hbm.at[idx_v.at[1]], uk_v, sem)
    cn = pltpu.async_copy(et_hbm.at[idx_v.at[2]], un_v, sem)
    cj.wait()
    ck.wait()
    cn.wait()

    def sample(s4, carry):
        for u in range(_UNROLL):
            s = s4 * _UNROLL + u
            acc = jnp.zeros((_L,), jnp.float32)
            for c in range(_CHK):
                uj = uj_v[s, pl.ds(c * _L, _L)]
                uk = uk_v[s, pl.ds(c * _L, _L)]
                un = un_v[s, pl.ds(c * _L, _L)]
                d = uj - un
                m = uj + un - uk - uk
                acc = acc + d * m
            part_v[s] = acc
        return carry

    lax.fori_loop(0, _BPW // _UNROLL, sample, jnp.int32(0))

    pltpu.sync_copy(part_v, out_hbm.at[pl.ds(base, _BPW)])


def _delta(et, idx):
    mesh = plsc.VectorSubcoreMesh(core_axis_name="c", subcore_axis_name="s")
    return pl.kernel(
        _delta_body,
        out_type=jax.ShapeDtypeStruct((_B, _L), jnp.float32),
        mesh=mesh,
        scratch_types=[
            pltpu.VMEM((3, _BPW), jnp.int32),
            pltpu.VMEM((_BPW, _D), jnp.float32),
            pltpu.VMEM((_BPW, _D), jnp.float32),
            pltpu.VMEM((_BPW, _D), jnp.float32),
            pltpu.VMEM((_BPW, _L), jnp.float32),
            pltpu.SemaphoreType.DMA,
        ],
        compiler_params=pltpu.CompilerParams(use_tc_tiling_on_sc=False,
                                             needs_layout_passes=False),
    )(et, idx)


_SROWS = 8


def _smooth_body(e_ref, l_ref, out_ref):
    i = pl.program_id(0)
    d = e_ref[...] - l_ref[...]
    s = jnp.sum(d * d)

    @pl.when(i == 0)
    def _():
        out_ref[0, 0] = s

    @pl.when(i > 0)
    def _():
        out_ref[0, 0] += s


def _smooth(e2, l2):
    grid = e2.shape[0] // _SROWS
    return pl.pallas_call(
        _smooth_body,
        grid=(grid,),
        in_specs=[
            pl.BlockSpec((_SROWS, _N), lambda i: (i, 0)),
            pl.BlockSpec((_SROWS, _N), lambda i: (i, 0)),
        ],
        out_specs=pl.BlockSpec(memory_space=pltpu.SMEM),
        out_shape=jax.ShapeDtypeStruct((1, 1), jnp.float32),
    )(e2, l2)


def _fin_body(dp_ref, sm_ref, out_ref):
    d = jnp.sum(dp_ref[...], axis=1)
    h = jnp.maximum(d + _MARGIN, 0.0)
    out_ref[0, 0] = jnp.sum(h) + _BETA1 * (float(_B) * sm_ref[0, 0])


def _fin(dp, sm):
    return pl.pallas_call(
        _fin_body,
        in_specs=[
            pl.BlockSpec((_B, _L), lambda: (0, 0)),
            pl.BlockSpec(memory_space=pltpu.SMEM),
        ],
        out_specs=pl.BlockSpec(memory_space=pltpu.SMEM),
        out_shape=jax.ShapeDtypeStruct((1, 1), jnp.float32),
    )(dp, sm)


@jax.jit
def kernel(embeddings, last_embeddings, triplets):
    idx = triplets.astype(jnp.int32).T
    dp = _delta(embeddings, idx)
    sm = _smooth(embeddings.T, last_embeddings.T)
    return _fin(dp, sm)[0, 0]
